# Initial kernel scaffold; baseline (speedup 1.0000x reference)
import jax
import jax.numpy as jnp
from jax.experimental import pallas as pl


def kernel(x, table0, table1, h0, h1):
    raise NotImplementedError("write your pallas kernel here")


# speedup vs baseline: 11.8593x; 11.8593x over previous
import functools

import jax
import jax.numpy as jnp
from jax import lax
from jax.experimental import pallas as pl
from jax.experimental.pallas import tpu as pltpu
from jax.experimental.pallas import tpu_sc as plsc

VOCAB = 100000
ROWS = 4096
CHUNK = 16
NCH = 4
BATCH = 16384

NC = 2
NS = 16
NW = NC * NS
B_PER_W = BATCH // NW
E_PER_W = B_PER_W * NCH
GCH = 128
NG = E_PER_W // GCH


def _body(x_hbm, h0_hbm, h1_hbm, t0_hbm, t1_hbm, out_hbm,
          xv, eidx, h0v, h1v, f0, f1, g0, g1, sem):
    wid = lax.axis_index("s") * NC + lax.axis_index("c")
    base_b = wid * B_PER_W
    base_e = wid * E_PER_W

    pltpu.sync_copy(x_hbm.at[pl.ds(base_b, B_PER_W)], xv)

    iota = lax.iota(jnp.int32, 16)
    cpat = lax.bitwise_and(iota, 3)

    def expand(j, carry):
        bl = lax.shift_right_logical(iota + j * 16, 1 + 1)
        v = plsc.load_gather(xv, [bl])
        r = lax.shift_right_logical(j, 3)
        k = lax.bitwise_and(j, 7)
        eidx[r, pl.ds(k * 16, 16)] = v * 4 + cpat
        return carry

    lax.fori_loop(0, NG * 8, expand, 0)

    h_copies = []
    for g in range(NG):
        h_copies.append(pltpu.async_copy(
            h0_hbm.at[eidx.at[g]], h0v.at[pl.ds(g * GCH, GCH)], sem))
        h_copies.append(pltpu.async_copy(
            h1_hbm.at[eidx.at[g]], h1v.at[pl.ds(g * GCH, GCH)], sem))
    for c in h_copies:
        c.wait()

    def rowids(j, carry):
        r = lax.shift_right_logical(j, 3)
        k = lax.bitwise_and(j, 7)
        f0[r, pl.ds(k * 16, 16)] = h0v[pl.ds(j * 16, 16)] * 4 + cpat
        f1[r, pl.ds(k * 16, 16)] = h1v[pl.ds(j * 16, 16)] * 4 + cpat
        return carry

    lax.fori_loop(0, NG * 8, rowids, 0)

    t_copies = []
    for g in range(NG):
        t_copies.append(pltpu.async_copy(
            t0_hbm.at[f0.at[g]], g0.at[pl.ds(g * GCH, GCH)], sem))
        t_copies.append(pltpu.async_copy(
            t1_hbm.at[f1.at[g]], g1.at[pl.ds(g * GCH, GCH)], sem))
    for c in t_copies:
        c.wait()

    def accum(i, carry):
        g0[i, :] = g0[i, :] + g1[i, :]
        return carry

    lax.fori_loop(0, E_PER_W, accum, 0)

    pltpu.sync_copy(g0, out_hbm.at[pl.ds(base_e, E_PER_W)])


@jax.jit
def _cc_embed(x, h0f, h1f, t0, t1):
    mesh = plsc.VectorSubcoreMesh(core_axis_name="c", subcore_axis_name="s")
    kfn = pl.kernel(
        _body,
        out_type=jax.ShapeDtypeStruct((BATCH * NCH, CHUNK), jnp.float32),
        mesh=mesh,
        compiler_params=pltpu.CompilerParams(
            needs_layout_passes=False, use_tc_tiling_on_sc=False),
        scratch_types=[
            pltpu.VMEM((B_PER_W,), jnp.int32),
            pltpu.VMEM((NG, GCH), jnp.int32),
            pltpu.VMEM((E_PER_W,), jnp.int32),
            pltpu.VMEM((E_PER_W,), jnp.int32),
            pltpu.VMEM((NG, GCH), jnp.int32),
            pltpu.VMEM((NG, GCH), jnp.int32),
            pltpu.VMEM((E_PER_W, CHUNK), jnp.float32),
            pltpu.VMEM((E_PER_W, CHUNK), jnp.float32),
            pltpu.SemaphoreType.DMA,
        ],
    )
    return kfn(x, h0f, h1f, t0, t1)


def kernel(x, table0, table1, h0, h1):
    h0f = h0.reshape(VOCAB * NCH)
    h1f = h1.reshape(VOCAB * NCH)
    t0 = table0.reshape(ROWS * NCH, CHUNK)
    t1 = table1.reshape(ROWS * NCH, CHUNK)
    out = _cc_embed(x.astype(jnp.int32), h0f, h1f, t0, t1)
    return out.reshape(BATCH, NCH * CHUNK)


# speedup vs baseline: 29.2185x; 2.4638x over previous
import jax
import jax.numpy as jnp
from jax import lax
from jax.experimental import pallas as pl
from jax.experimental.pallas import tpu as pltpu
from jax.experimental.pallas import tpu_sc as plsc

VOCAB = 100000
ROWS = 4096
CHUNK = 16
NCH = 4
BATCH = 16384

NC = 2
NS = 16
NW = NC * NS
B_PER_W = BATCH // NW
E_PER_W = B_PER_W * NCH
GCH = 128
NB = B_PER_W // GCH


def _body(x_hbm, h0_hbm, h1_hbm, t0_hbm, t1_hbm, out_hbm,
          xv, e0, e1, h0v, h1v, g0, g1, sem):
    wid = lax.axis_index("s") * NC + lax.axis_index("c")
    base_b = wid * B_PER_W

    pltpu.sync_copy(x_hbm.at[pl.ds(base_b, B_PER_W)], xv)

    def eidx(j, carry):
        c = lax.shift_right_logical(j, 5)
        i = lax.bitwise_and(j, 31)
        v = xv[pl.ds(i * 16, 16)]
        e0[c, pl.ds(i * 16, 16)] = v + c * VOCAB
        return carry

    lax.fori_loop(0, NCH * (B_PER_W // 16), eidx, 0)

    h_copies = []
    for c in range(NCH):
        for b in range(NB):
            sl = pl.ds(b * GCH, GCH)
            h_copies.append(pltpu.async_copy(
                h0_hbm.at[e0.at[c, sl]], h0v.at[c, sl], sem))
            h_copies.append(pltpu.async_copy(
                h1_hbm.at[e0.at[c, sl]], h1v.at[c, sl], sem))
    for cp in h_copies:
        cp.wait()

    def rowids(j, carry):
        c = lax.shift_right_logical(j, 5)
        i = lax.bitwise_and(j, 31)
        sl = pl.ds(i * 16, 16)
        e0[c, sl] = h0v[c, sl] * 4 + c
        e1[c, sl] = h1v[c, sl] * 4 + c
        return carry

    lax.fori_loop(0, NCH * (B_PER_W // 16), rowids, 0)

    t_copies = []
    for c in range(NCH):
        for b in range(NB):
            sl = pl.ds(b * GCH, GCH)
            row0 = (c * NB + b) * GCH
            t_copies.append(pltpu.async_copy(
                t0_hbm.at[e0.at[c, sl]], g0.at[pl.ds(row0, GCH)], sem))
            t_copies.append(pltpu.async_copy(
                t1_hbm.at[e1.at[c, sl]], g1.at[pl.ds(row0, GCH)], sem))
    for cp in t_copies:
        cp.wait()

    def accum(i, carry):
        g0[i, :] = g0[i, :] + g1[i, :]
        return carry

    lax.fori_loop(0, E_PER_W, accum, 0)

    for c in range(NCH):
        pltpu.sync_copy(
            g0.at[pl.ds(c * B_PER_W, B_PER_W)],
            out_hbm.at[pl.ds(base_b, B_PER_W), pl.ds(c * CHUNK, CHUNK)])


@jax.jit
def _cc_embed(x, h0t, h1t, t0, t1):
    mesh = plsc.VectorSubcoreMesh(core_axis_name="c", subcore_axis_name="s")
    kfn = pl.kernel(
        _body,
        out_type=jax.ShapeDtypeStruct((BATCH, NCH * CHUNK), jnp.float32),
        mesh=mesh,
        compiler_params=pltpu.CompilerParams(
            needs_layout_passes=False, use_tc_tiling_on_sc=False),
        scratch_types=[
            pltpu.VMEM((B_PER_W,), jnp.int32),
            pltpu.VMEM((NCH, B_PER_W), jnp.int32),
            pltpu.VMEM((NCH, B_PER_W), jnp.int32),
            pltpu.VMEM((NCH, B_PER_W), jnp.int32),
            pltpu.VMEM((NCH, B_PER_W), jnp.int32),
            pltpu.VMEM((E_PER_W, CHUNK), jnp.float32),
            pltpu.VMEM((E_PER_W, CHUNK), jnp.float32),
            pltpu.SemaphoreType.DMA,
        ],
    )
    return kfn(x, h0t, h1t, t0, t1)


def kernel(x, table0, table1, h0, h1):
    h0t = h0.T.reshape(VOCAB * NCH)
    h1t = h1.T.reshape(VOCAB * NCH)
    t0 = table0.reshape(ROWS * NCH, CHUNK)
    t1 = table1.reshape(ROWS * NCH, CHUNK)
    return _cc_embed(x.astype(jnp.int32), h0t, h1t, t0, t1)


# speedup vs baseline: 31.9690x; 1.0941x over previous
import jax
import jax.numpy as jnp
from jax import lax
from jax.experimental import pallas as pl
from jax.experimental.pallas import tpu as pltpu
from jax.experimental.pallas import tpu_sc as plsc

VOCAB = 100000
ROWS = 4096
CHUNK = 16
NCH = 4
BATCH = 16384

NC = 2
NS = 16
NW = NC * NS
B_PER_W = BATCH // NW
E_PER_W = B_PER_W * NCH
GCH = 128
NB = B_PER_W // GCH


def _body(x_hbm, h0_hbm, h1_hbm, t0_hbm, t1_hbm, out_hbm,
          xv, e0, e1, h0v, h1v, g0, g1, sem):
    wid = lax.axis_index("s") * NC + lax.axis_index("c")
    base_b = wid * B_PER_W

    pltpu.sync_copy(x_hbm.at[pl.ds(base_b, B_PER_W)], xv)

    @plsc.parallel_loop(0, NCH * (B_PER_W // 16), unroll=4)
    def _(j):
        c = lax.shift_right_logical(j, 5)
        i = lax.bitwise_and(j, 31)
        v = xv[pl.ds(i * 16, 16)]
        e0[c, pl.ds(i * 16, 16)] = v + c * VOCAB

    h_copies = []
    for c in range(NCH):
        for b in range(NB):
            sl = pl.ds(b * GCH, GCH)
            h_copies.append(pltpu.async_copy(
                h0_hbm.at[e0.at[c, sl]], h0v.at[c, sl], sem))
            h_copies.append(pltpu.async_copy(
                h1_hbm.at[e0.at[c, sl]], h1v.at[c, sl], sem))
    for cp in h_copies:
        cp.wait()

    @plsc.parallel_loop(0, NCH * (B_PER_W // 16), unroll=4)
    def _(j):
        c = lax.shift_right_logical(j, 5)
        i = lax.bitwise_and(j, 31)
        sl = pl.ds(i * 16, 16)
        e0[c, sl] = h0v[c, sl] * 4 + c
        e1[c, sl] = h1v[c, sl] * 4 + c

    t_copies = []
    for c in range(NCH):
        for b in range(NB):
            sl = pl.ds(b * GCH, GCH)
            row0 = (c * NB + b) * GCH
            t_copies.append(pltpu.async_copy(
                t0_hbm.at[e0.at[c, sl]], g0.at[pl.ds(row0, GCH)], sem))
            t_copies.append(pltpu.async_copy(
                t1_hbm.at[e1.at[c, sl]], g1.at[pl.ds(row0, GCH)], sem))
    for cp in t_copies:
        cp.wait()

    @plsc.parallel_loop(0, E_PER_W, unroll=8)
    def _(i):
        g0[i, :] = g0[i, :] + g1[i, :]

    for c in range(NCH):
        pltpu.sync_copy(
            g0.at[pl.ds(c * B_PER_W, B_PER_W)],
            out_hbm.at[pl.ds(base_b, B_PER_W), pl.ds(c * CHUNK, CHUNK)])


@jax.jit
def _cc_embed(x, h0t, h1t, t0, t1):
    mesh = plsc.VectorSubcoreMesh(core_axis_name="c", subcore_axis_name="s")
    kfn = pl.kernel(
        _body,
        out_type=jax.ShapeDtypeStruct((BATCH, NCH * CHUNK), jnp.float32),
        mesh=mesh,
        compiler_params=pltpu.CompilerParams(
            needs_layout_passes=False, use_tc_tiling_on_sc=False),
        scratch_types=[
            pltpu.VMEM((B_PER_W,), jnp.int32),
            pltpu.VMEM((NCH, B_PER_W), jnp.int32),
            pltpu.VMEM((NCH, B_PER_W), jnp.int32),
            pltpu.VMEM((NCH, B_PER_W), jnp.int32),
            pltpu.VMEM((NCH, B_PER_W), jnp.int32),
            pltpu.VMEM((E_PER_W, CHUNK), jnp.float32),
            pltpu.VMEM((E_PER_W, CHUNK), jnp.float32),
            pltpu.SemaphoreType.DMA,
        ],
    )
    return kfn(x, h0t, h1t, t0, t1)


def kernel(x, table0, table1, h0, h1):
    h0t = h0.T.reshape(VOCAB * NCH)
    h1t = h1.T.reshape(VOCAB * NCH)
    t0 = table0.reshape(ROWS * NCH, CHUNK)
    t1 = table1.reshape(ROWS * NCH, CHUNK)
    return _cc_embed(x.astype(jnp.int32), h0t, h1t, t0, t1)


# speedup vs baseline: 32.9806x; 1.0316x over previous
import jax
import jax.numpy as jnp
from jax import lax
from jax.experimental import pallas as pl
from jax.experimental.pallas import tpu as pltpu
from jax.experimental.pallas import tpu_sc as plsc

VOCAB = 100000
ROWS = 4096
CHUNK = 16
NCH = 4
BATCH = 16384

NC = 2
NS = 16
NW = NC * NS
B_PER_W = BATCH // NW
E_PER_W = B_PER_W * NCH
GCH = 128
NB = B_PER_W // GCH


def _body(x_hbm, h0_hbm, h1_hbm, t0_hbm, t1_hbm, out_hbm,
          xv, e0, e1, h0v, h1v, g0, g1, sem):
    wid = lax.axis_index("s") * NC + lax.axis_index("c")
    base_b = wid * B_PER_W

    pltpu.sync_copy(x_hbm.at[pl.ds(base_b, B_PER_W)], xv)

    @plsc.parallel_loop(0, NCH * (B_PER_W // 16), unroll=4)
    def _(j):
        c = lax.shift_right_logical(j, 5)
        i = lax.bitwise_and(j, 31)
        v = xv[pl.ds(i * 16, 16)]
        e0[c, pl.ds(i * 16, 16)] = v + c * VOCAB

    h_copies = []
    for c in range(NCH):
        for b in range(NB):
            sl = pl.ds(b * GCH, GCH)
            h_copies.append(pltpu.async_copy(
                h0_hbm.at[e0.at[c, sl]], h0v.at[c, sl], sem))
            h_copies.append(pltpu.async_copy(
                h1_hbm.at[e0.at[c, sl]], h1v.at[c, sl], sem))
    for cp in h_copies:
        cp.wait()

    @plsc.parallel_loop(0, NCH * (B_PER_W // 16), unroll=4)
    def _(j):
        c = lax.shift_right_logical(j, 5)
        i = lax.bitwise_and(j, 31)
        sl = pl.ds(i * 16, 16)
        e0[c, sl] = h0v[c, sl] + c * ROWS
        e1[c, sl] = h1v[c, sl] + c * ROWS

    t_copies = []
    for c in range(NCH):
        for b in range(NB):
            sl = pl.ds(b * GCH, GCH)
            row0 = (c * NB + b) * GCH
            t_copies.append(pltpu.async_copy(
                t0_hbm.at[e0.at[c, sl]], g0.at[pl.ds(row0, GCH)], sem))
            t_copies.append(pltpu.async_copy(
                t1_hbm.at[e1.at[c, sl]], g1.at[pl.ds(row0, GCH)], sem))
    for cp in t_copies:
        cp.wait()

    @plsc.parallel_loop(0, E_PER_W, unroll=8)
    def _(i):
        g0[i, :] = g0[i, :] + g1[i, :]

    for c in range(NCH):
        pltpu.sync_copy(
            g0.at[pl.ds(c * B_PER_W, B_PER_W)],
            out_hbm.at[pl.ds(base_b, B_PER_W), pl.ds(c * CHUNK, CHUNK)])


@jax.jit
def _cc_embed(x, h0t, h1t, t0, t1):
    mesh = plsc.VectorSubcoreMesh(core_axis_name="c", subcore_axis_name="s")
    kfn = pl.kernel(
        _body,
        out_type=jax.ShapeDtypeStruct((BATCH, NCH * CHUNK), jnp.float32),
        mesh=mesh,
        compiler_params=pltpu.CompilerParams(
            needs_layout_passes=False, use_tc_tiling_on_sc=False),
        scratch_types=[
            pltpu.VMEM((B_PER_W,), jnp.int32),
            pltpu.VMEM((NCH, B_PER_W), jnp.int32),
            pltpu.VMEM((NCH, B_PER_W), jnp.int32),
            pltpu.VMEM((NCH, B_PER_W), jnp.int32),
            pltpu.VMEM((NCH, B_PER_W), jnp.int32),
            pltpu.VMEM((E_PER_W, CHUNK), jnp.float32),
            pltpu.VMEM((E_PER_W, CHUNK), jnp.float32),
            pltpu.SemaphoreType.DMA,
        ],
    )
    return kfn(x, h0t, h1t, t0, t1)


def kernel(x, table0, table1, h0, h1):
    h0t = h0.T.reshape(VOCAB * NCH)
    h1t = h1.T.reshape(VOCAB * NCH)
    t0 = table0.transpose(1, 0, 2).reshape(NCH * ROWS, CHUNK)
    t1 = table1.transpose(1, 0, 2).reshape(NCH * ROWS, CHUNK)
    return _cc_embed(x.astype(jnp.int32), h0t, h1t, t0, t1)


# speedup vs baseline: 35.8524x; 1.0871x over previous
import jax
import jax.numpy as jnp
from jax import lax
from jax.experimental import pallas as pl
from jax.experimental.pallas import tpu as pltpu
from jax.experimental.pallas import tpu_sc as plsc

VOCAB = 100000
ROWS = 4096
CHUNK = 16
NCH = 4
BATCH = 16384

NC = 2
NS = 16
NW = NC * NS
B_PER_W = BATCH // NW
E_PER_W = B_PER_W * NCH
GCH = 128
NB = B_PER_W // GCH
R_PER_T = ROWS // NS
HB = 128
HB_LOG = 7


def _body(x_hbm, h0_hbm, h1_hbm, t0_hbm, t1_hbm, out_hbm,
          xv, e0, e1, h0v, h1v, slab, tbuf, ts0, ts1, g0, g1, sem):
    sid = lax.axis_index("s")
    wid = sid * NC + lax.axis_index("c")
    base_b = wid * B_PER_W

    pltpu.sync_copy(x_hbm.at[pl.ds(base_b, B_PER_W)], xv)

    @plsc.parallel_loop(0, NCH * (B_PER_W // 16), unroll=4)
    def _(j):
        c = lax.shift_right_logical(j, 5)
        i = lax.bitwise_and(j, 31)
        e0[c, pl.ds(i * 16, 16)] = xv[pl.ds(i * 16, 16)] + c * VOCAB

    h_copies = []
    for c in range(NCH):
        for b in range(NB):
            sl = pl.ds(b * GCH, GCH)
            h_copies.append(pltpu.async_copy(
                h0_hbm.at[e0.at[c, sl]], h0v.at[c, sl], sem))
            h_copies.append(pltpu.async_copy(
                h1_hbm.at[e0.at[c, sl]], h1v.at[c, sl], sem))

    iota = lax.iota(jnp.int32, 16)
    for t_hbm, ts in ((t0_hbm, ts0), (t1_hbm, ts1)):
        for half in range(R_PER_T // HB):
            col0 = sid * R_PER_T + half * HB
            pltpu.sync_copy(t_hbm.at[:, pl.ds(col0, HB)], slab)

            @plsc.parallel_loop(0, NCH * HB, unroll=4)
            def _(j):
                c = lax.shift_right_logical(j, HB_LOG)
                rl = lax.bitwise_and(j, HB - 1)
                tbuf[j, :] = plsc.load_gather(
                    slab, [c * CHUNK + iota, jnp.full((16,), 0, jnp.int32) + rl])

            for c in range(NCH):
                pltpu.sync_copy(
                    tbuf.at[pl.ds(c * HB, HB)],
                    ts.at[pl.ds(c * ROWS + col0, HB)])

    for cp in h_copies:
        cp.wait()

    @plsc.parallel_loop(0, NCH * (B_PER_W // 16), unroll=4)
    def _(j):
        c = lax.shift_right_logical(j, 5)
        i = lax.bitwise_and(j, 31)
        sl = pl.ds(i * 16, 16)
        e0[c, sl] = h0v[c, sl] + c * ROWS
        e1[c, sl] = h1v[c, sl] + c * ROWS

    plsc.subcore_barrier()

    t_copies = []
    for c in range(NCH):
        for b in range(NB):
            sl = pl.ds(b * GCH, GCH)
            row0 = (c * NB + b) * GCH
            t_copies.append(pltpu.async_copy(
                ts0.at[e0.at[c, sl]], g0.at[pl.ds(row0, GCH)], sem))
            t_copies.append(pltpu.async_copy(
                ts1.at[e1.at[c, sl]], g1.at[pl.ds(row0, GCH)], sem))
    for cp in t_copies:
        cp.wait()

    @plsc.parallel_loop(0, E_PER_W, unroll=8)
    def _(i):
        g0[i, :] = g0[i, :] + g1[i, :]

    for c in range(NCH):
        pltpu.sync_copy(
            g0.at[pl.ds(c * B_PER_W, B_PER_W)],
            out_hbm.at[pl.ds(base_b, B_PER_W), pl.ds(c * CHUNK, CHUNK)])


@jax.jit
def _cc_embed(x, h0t, h1t, t0, t1):
    mesh = plsc.VectorSubcoreMesh(core_axis_name="c", subcore_axis_name="s")
    kfn = pl.kernel(
        _body,
        out_type=jax.ShapeDtypeStruct((BATCH, NCH * CHUNK), jnp.float32),
        mesh=mesh,
        compiler_params=pltpu.CompilerParams(
            needs_layout_passes=False, use_tc_tiling_on_sc=False),
        scratch_types=[
            pltpu.VMEM((B_PER_W,), jnp.int32),
            pltpu.VMEM((NCH, B_PER_W), jnp.int32),
            pltpu.VMEM((NCH, B_PER_W), jnp.int32),
            pltpu.VMEM((NCH, B_PER_W), jnp.int32),
            pltpu.VMEM((NCH, B_PER_W), jnp.int32),
            pltpu.VMEM((NCH * CHUNK, HB), jnp.float32),
            pltpu.VMEM((NCH * HB, CHUNK), jnp.float32),
            pltpu.VMEM_SHARED((NCH * ROWS, CHUNK), jnp.float32),
            pltpu.VMEM_SHARED((NCH * ROWS, CHUNK), jnp.float32),
            pltpu.VMEM((E_PER_W, CHUNK), jnp.float32),
            pltpu.VMEM((E_PER_W, CHUNK), jnp.float32),
            pltpu.SemaphoreType.DMA,
        ],
    )
    return kfn(x, h0t, h1t, t0, t1)


def kernel(x, table0, table1, h0, h1):
    h0t = h0.T.reshape(VOCAB * NCH)
    h1t = h1.T.reshape(VOCAB * NCH)
    t0 = table0.transpose(1, 2, 0).reshape(NCH * CHUNK, ROWS)
    t1 = table1.transpose(1, 2, 0).reshape(NCH * CHUNK, ROWS)
    return _cc_embed(x.astype(jnp.int32), h0t, h1t, t0, t1)
